# Initial kernel scaffold; baseline (speedup 1.0000x reference)
import jax
import jax.numpy as jnp
from jax.experimental import pallas as pl


def kernel(label, feat, centers):
    raise NotImplementedError("write your pallas kernel here")


# speedup vs baseline: 1.6259x; 1.6259x over previous
import functools

import jax
import jax.numpy as jnp
from jax import lax
from jax.experimental import pallas as pl
from jax.experimental.pallas import tpu as pltpu
from jax.experimental.pallas import tpu_sc as plsc

NUM_CLASSES = 1000
FEAT_DIM = 128
BATCH = 16384
CURVATURE = 1.0

_NC = 2
_NS = 16
_NW = _NC * _NS
_BPW = BATCH // _NW
_CHUNK = 128
_NCHUNK = _BPW // _CHUNK


def _sc_gather_body(centers_hbm, label_hbm, out_hbm, idx_v, rows_v, sem):
    wid = lax.axis_index("s") * _NC + lax.axis_index("c")
    pltpu.sync_copy(label_hbm.at[wid], idx_v)
    copies = []
    for j in range(_NCHUNK):
        copies.append(
            pltpu.async_copy(centers_hbm.at[idx_v.at[j]], rows_v.at[j], sem))
    for c in copies:
        c.wait()
    pltpu.sync_copy(rows_v, out_hbm.at[wid])


@functools.partial(jax.jit, static_argnames=())
def _sc_gather(centers, label3d):
    mesh = plsc.VectorSubcoreMesh(core_axis_name="c", subcore_axis_name="s")
    fn = pl.kernel(
        _sc_gather_body,
        out_type=jax.ShapeDtypeStruct((_NW, _NCHUNK, _CHUNK, FEAT_DIM),
                                      jnp.float32),
        mesh=mesh,
        scratch_types=[
            pltpu.VMEM((_NCHUNK, _CHUNK), jnp.int32),
            pltpu.VMEM((_NCHUNK, _CHUNK, FEAT_DIM), jnp.float32),
            pltpu.SemaphoreType.DMA,
        ],
    )
    return fn(centers, label3d)


_TC_TILE = 1024
_TC_STEPS = BATCH // _TC_TILE


def _tc_loss_body(f_ref, g_ref, out_ref):
    i = pl.program_id(0)
    f = f_ref[...]
    g = g_ref[...]
    x2 = jnp.sum(f * f, axis=1, keepdims=True)
    y2 = jnp.sum(g * g, axis=1, keepdims=True)
    xy = jnp.sum(f * g, axis=1, keepdims=True)
    c = jnp.float32(CURVATURE)
    denom = 1.0 + c * x2 * y2 - 2.0 * c * xy + 1e-08
    md = ((1.0 + c * y2) * (-f) + (1.0 - c * x2) * g) / denom
    norm = jnp.sqrt(jnp.sum(md * md, axis=1, keepdims=True))
    sqrt_c = jnp.sqrt(c)
    z = jnp.clip(sqrt_c * norm, 1e-08, 1.0 - 1e-05)
    atanh = 0.5 * jnp.log((1.0 + z) / (1.0 - z))
    dist = (2.0 / sqrt_c) * atanh
    s = jnp.sum(dist)

    @pl.when(i == 0)
    def _():
        out_ref[0, 0] = 0.0

    out_ref[0, 0] += s

    @pl.when(i == _TC_STEPS - 1)
    def _():
        out_ref[0, 0] = out_ref[0, 0] * jnp.float32(1.0 / BATCH)


def _tc_loss(feat, gathered):
    return pl.pallas_call(
        _tc_loss_body,
        grid=(_TC_STEPS,),
        in_specs=[
            pl.BlockSpec((_TC_TILE, FEAT_DIM), lambda i: (i, 0)),
            pl.BlockSpec((_TC_TILE, FEAT_DIM), lambda i: (i, 0)),
        ],
        out_specs=pl.BlockSpec(memory_space=pltpu.SMEM),
        out_shape=jax.ShapeDtypeStruct((1, 1), jnp.float32),
    )(feat, gathered)


def kernel(label, feat, centers):
    label3d = label.astype(jnp.int32).reshape(_NW, _NCHUNK, _CHUNK)
    gathered = _sc_gather(centers, label3d).reshape(BATCH, FEAT_DIM)
    loss = _tc_loss(feat, gathered)
    return loss[0, 0]


# speedup vs baseline: 1.7219x; 1.0590x over previous
import jax
import jax.numpy as jnp
from jax import lax
from jax.experimental import pallas as pl
from jax.experimental.pallas import tpu as pltpu
from jax.experimental.pallas import tpu_sc as plsc

NUM_CLASSES = 1000
FEAT_DIM = 128
BATCH = 16384
CURVATURE = 1.0

_NC = 2
_NS = 16
_NW = _NC * _NS
_BPW = BATCH // _NW
_CH = 128
_NCH = _BPW // _CH
_GRP = _CH // 16
_L = 16


def _sc_fused_body(feat_hbm, centers_hbm, label_hbm,
                   dot_hbm, x2_hbm, y2_hbm,
                   idx_v, fbuf0, fbuf1, cbuf0, cbuf1,
                   dot_v, x2_v, y2_v,
                   semf0, semf1, semc0, semc1):
    wid = lax.axis_index("s") * _NC + lax.axis_index("c")
    pltpu.sync_copy(label_hbm.at[wid], idx_v)

    fbufs = (fbuf0, fbuf1)
    cbufs = (cbuf0, cbuf1)
    semfs = (semf0, semf1)
    semcs = (semc0, semc1)

    def issue(j):
        p = j & 1
        cf = pltpu.async_copy(feat_hbm.at[wid, j], fbufs[p], semfs[p])
        cc = pltpu.async_copy(centers_hbm.at[idx_v.at[j]], cbufs[p], semcs[p])
        return (cf, cc)

    pending = {0: issue(0)}
    for j in range(_NCH):
        if j + 1 < _NCH:
            pending[j + 1] = issue(j + 1)
        for c in pending.pop(j):
            c.wait()
        p = j & 1
        fb = fbufs[p]
        cb = cbufs[p]

        def group_body(g, _, fb=fb, cb=cb, j=j):
            for rr in range(_L):
                r = g * _L + rr
                dot = jnp.zeros((_L,), jnp.float32)
                x2 = jnp.zeros((_L,), jnp.float32)
                y2 = jnp.zeros((_L,), jnp.float32)
                for q in range(FEAT_DIM // _L):
                    f = fb[r, pl.ds(q * _L, _L)]
                    c = cb[r, pl.ds(q * _L, _L)]
                    dot = dot + f * c
                    x2 = x2 + f * f
                    y2 = y2 + c * c
                off = (j * _CH + r) * _L
                dot_v[pl.ds(off, _L)] = dot
                x2_v[pl.ds(off, _L)] = x2
                y2_v[pl.ds(off, _L)] = y2
            return 0

        lax.fori_loop(0, _GRP, group_body, 0)

    pltpu.sync_copy(dot_v, dot_hbm.at[wid])
    pltpu.sync_copy(x2_v, x2_hbm.at[wid])
    pltpu.sync_copy(y2_v, y2_hbm.at[wid])


def _sc_fused(feat4, centers, label3):
    mesh = plsc.VectorSubcoreMesh(core_axis_name="c", subcore_axis_name="s")
    out = jax.ShapeDtypeStruct((_NW, _BPW * _L), jnp.float32)
    fn = pl.kernel(
        _sc_fused_body,
        out_type=(out, out, out),
        mesh=mesh,
        scratch_types=[
            pltpu.VMEM((_NCH, _CH), jnp.int32),
            pltpu.VMEM((_CH, FEAT_DIM), jnp.float32),
            pltpu.VMEM((_CH, FEAT_DIM), jnp.float32),
            pltpu.VMEM((_CH, FEAT_DIM), jnp.float32),
            pltpu.VMEM((_CH, FEAT_DIM), jnp.float32),
            pltpu.VMEM((_BPW * _L,), jnp.float32),
            pltpu.VMEM((_BPW * _L,), jnp.float32),
            pltpu.VMEM((_BPW * _L,), jnp.float32),
            pltpu.SemaphoreType.DMA,
            pltpu.SemaphoreType.DMA,
            pltpu.SemaphoreType.DMA,
            pltpu.SemaphoreType.DMA,
        ],
    )
    return fn(feat4, centers, label3)


_RK = BATCH * _L // 128


def _tc_finish_body(dp_ref, xp_ref, yp_ref, out_ref):
    ii = lax.broadcasted_iota(jnp.int32, (_RK, 128), 0)
    jj = lax.broadcasted_iota(jnp.int32, (_RK, 128), 1)
    bsel = (ii // _L == jj).astype(jnp.float32)
    dot = jnp.dot(dp_ref[...], bsel, preferred_element_type=jnp.float32)
    x2 = jnp.dot(xp_ref[...], bsel, preferred_element_type=jnp.float32)
    y2 = jnp.dot(yp_ref[...], bsel, preferred_element_type=jnp.float32)
    c = jnp.float32(CURVATURE)
    denom = 1.0 + c * x2 * y2 - 2.0 * c * dot + 1e-08
    a = 1.0 + c * y2
    b = 1.0 - c * x2
    num2 = a * a * x2 + b * b * y2 - 2.0 * a * b * dot
    num2 = jnp.maximum(num2, 0.0)
    sqrt_c = jnp.sqrt(c)
    norm = jnp.sqrt(num2) / denom
    z = jnp.clip(sqrt_c * norm, 1e-08, 1.0 - 1e-05)
    dist = (1.0 / sqrt_c) * jnp.log((1.0 + z) / (1.0 - z))
    out_ref[0, 0] = jnp.sum(dist) * jnp.float32(1.0 / BATCH)


def _tc_finish(dp2, xp2, yp2):
    return pl.pallas_call(
        _tc_finish_body,
        out_specs=pl.BlockSpec(memory_space=pltpu.SMEM),
        out_shape=jax.ShapeDtypeStruct((1, 1), jnp.float32),
    )(dp2, xp2, yp2)


def kernel(label, feat, centers):
    label3 = label.astype(jnp.int32).reshape(_NW, _NCH, _CH)
    feat4 = feat.reshape(_NW, _NCH, _CH, FEAT_DIM)
    dp, xp, yp = _sc_fused(feat4, centers, label3)
    loss = _tc_finish(dp.reshape(128, _RK), xp.reshape(128, _RK),
                      yp.reshape(128, _RK))
    return loss[0, 0]


# speedup vs baseline: 2.1604x; 1.2547x over previous
import jax
import jax.numpy as jnp
from jax import lax
from jax.experimental import pallas as pl
from jax.experimental.pallas import tpu as pltpu
from jax.experimental.pallas import tpu_sc as plsc

NUM_CLASSES = 1000
FEAT_DIM = 128
BATCH = 16384
CURVATURE = 1.0

_NC = 2
_NS = 16
_NW = _NC * _NS
_BPW = BATCH // _NW
_CH = 128
_NCH = _BPW // _CH
_L = 16
_RU = 4
_QR = FEAT_DIM // _L
_PCH = _CH * _L


def _sc_fused_body(feat_hbm, centers_hbm, label_hbm,
                   dot_hbm, x2_hbm, y2_hbm,
                   idx_v, fbuf0, fbuf1, cbuf0, cbuf1,
                   dot_v, x2_v, y2_v,
                   semf0, semf1, semc0, semc1, semo):
    wid = lax.axis_index("s") * _NC + lax.axis_index("c")
    pltpu.sync_copy(label_hbm.at[wid], idx_v)

    fbufs = (fbuf0, fbuf1)
    cbufs = (cbuf0, cbuf1)
    semfs = (semf0, semf1)
    semcs = (semc0, semc1)

    def issue(j):
        p = j & 1
        cf = pltpu.async_copy(feat_hbm.at[wid, j], fbufs[p], semfs[p])
        cc = pltpu.async_copy(centers_hbm.at[idx_v.at[j]], cbufs[p], semcs[p])
        return (cf, cc)

    out_row = wid * _NCH
    pending = {0: issue(0)}
    outcopies = []
    for j in range(_NCH):
        if j + 1 < _NCH:
            pending[j + 1] = issue(j + 1)
        for c in pending.pop(j):
            c.wait()
        p = j & 1
        fb = fbufs[p]
        cb = cbufs[p]

        def blk_body(b, _, fb=fb, cb=cb, j=j):
            for rr in range(_RU):
                r = b * _RU + rr
                dot0 = jnp.zeros((_L,), jnp.float32)
                dot1 = jnp.zeros((_L,), jnp.float32)
                x20 = jnp.zeros((_L,), jnp.float32)
                x21 = jnp.zeros((_L,), jnp.float32)
                y20 = jnp.zeros((_L,), jnp.float32)
                y21 = jnp.zeros((_L,), jnp.float32)
                for q in range(_QR):
                    f = fb[r, pl.ds(q * _L, _L)]
                    c = cb[r, pl.ds(q * _L, _L)]
                    if q & 1:
                        dot1 = dot1 + f * c
                        x21 = x21 + f * f
                        y21 = y21 + c * c
                    else:
                        dot0 = dot0 + f * c
                        x20 = x20 + f * f
                        y20 = y20 + c * c
                off = (j * _CH + r) * _L
                dot_v[pl.ds(off, _L)] = dot0 + dot1
                x2_v[pl.ds(off, _L)] = x20 + x21
                y2_v[pl.ds(off, _L)] = y20 + y21
            return 0

        lax.fori_loop(0, _CH // _RU, blk_body, 0)

        sl = pl.ds(j * _PCH, _PCH)
        outcopies.append(
            pltpu.async_copy(dot_v.at[sl], dot_hbm.at[out_row + j], semo))
        outcopies.append(
            pltpu.async_copy(x2_v.at[sl], x2_hbm.at[out_row + j], semo))
        outcopies.append(
            pltpu.async_copy(y2_v.at[sl], y2_hbm.at[out_row + j], semo))
    for c in outcopies:
        c.wait()


def _sc_fused(feat4, centers, label3):
    mesh = plsc.VectorSubcoreMesh(core_axis_name="c", subcore_axis_name="s")
    out = jax.ShapeDtypeStruct((_NW * _NCH, _PCH), jnp.float32)
    fn = pl.kernel(
        _sc_fused_body,
        out_type=(out, out, out),
        mesh=mesh,
        scratch_types=[
            pltpu.VMEM((_NCH, _CH), jnp.int32),
            pltpu.VMEM((_CH, FEAT_DIM), jnp.float32),
            pltpu.VMEM((_CH, FEAT_DIM), jnp.float32),
            pltpu.VMEM((_CH, FEAT_DIM), jnp.float32),
            pltpu.VMEM((_CH, FEAT_DIM), jnp.float32),
            pltpu.VMEM((_BPW * _L,), jnp.float32),
            pltpu.VMEM((_BPW * _L,), jnp.float32),
            pltpu.VMEM((_BPW * _L,), jnp.float32),
            pltpu.SemaphoreType.DMA,
            pltpu.SemaphoreType.DMA,
            pltpu.SemaphoreType.DMA,
            pltpu.SemaphoreType.DMA,
            pltpu.SemaphoreType.DMA,
        ],
    )
    return fn(feat4, centers, label3)


_RK = BATCH * _L // 128


def _tc_finish_body(dp_ref, xp_ref, yp_ref, out_ref):
    ii = lax.broadcasted_iota(jnp.int32, (_RK, 128), 0)
    jj = lax.broadcasted_iota(jnp.int32, (_RK, 128), 1)
    bsel = (ii // _L == jj).astype(jnp.float32)
    dot = jnp.dot(dp_ref[...], bsel, preferred_element_type=jnp.float32)
    x2 = jnp.dot(xp_ref[...], bsel, preferred_element_type=jnp.float32)
    y2 = jnp.dot(yp_ref[...], bsel, preferred_element_type=jnp.float32)
    c = jnp.float32(CURVATURE)
    denom = 1.0 + c * x2 * y2 - 2.0 * c * dot + 1e-08
    a = 1.0 + c * y2
    b = 1.0 - c * x2
    num2 = a * a * x2 + b * b * y2 - 2.0 * a * b * dot
    num2 = jnp.maximum(num2, 0.0)
    sqrt_c = jnp.sqrt(c)
    norm = jnp.sqrt(num2) / denom
    z = jnp.clip(sqrt_c * norm, 1e-08, 1.0 - 1e-05)
    dist = (1.0 / sqrt_c) * jnp.log((1.0 + z) / (1.0 - z))
    out_ref[0, 0] = jnp.sum(dist) * jnp.float32(1.0 / BATCH)


def _tc_finish(dp2, xp2, yp2):
    return pl.pallas_call(
        _tc_finish_body,
        out_specs=pl.BlockSpec(memory_space=pltpu.SMEM),
        out_shape=jax.ShapeDtypeStruct((1, 1), jnp.float32),
    )(dp2, xp2, yp2)


def kernel(label, feat, centers):
    label3 = label.astype(jnp.int32).reshape(_NW, _NCH, _CH)
    feat4 = feat.reshape(_NW, _NCH, _CH, FEAT_DIM)
    dp, xp, yp = _sc_fused(feat4, centers, label3)
    loss = _tc_finish(dp, xp, yp)
    return loss[0, 0]


# speedup vs baseline: 2.2958x; 1.0627x over previous
import jax
import jax.numpy as jnp
from jax import lax
from jax.experimental import pallas as pl
from jax.experimental.pallas import tpu as pltpu
from jax.experimental.pallas import tpu_sc as plsc

NUM_CLASSES = 1000
FEAT_DIM = 128
BATCH = 16384
CURVATURE = 1.0

_NC = 2
_NS = 16
_NW = _NC * _NS
_BPW = BATCH // _NW
_CH = 128
_NCH = _BPW // _CH
_L = 16
_RU = 4
_QR = FEAT_DIM // _L
_PCH = _CH * _L


def _sc_fused_body(feat_hbm, centers_hbm, label_hbm,
                   dot_hbm, x2_hbm, y2_hbm,
                   idx_v, fbuf0, fbuf1, cbuf0, cbuf1,
                   dot_v, x2_v, y2_v,
                   semf0, semf1, semc0, semc1, semo):
    wid = lax.axis_index("s") * _NC + lax.axis_index("c")
    pltpu.sync_copy(label_hbm.at[wid], idx_v)

    fbufs = (fbuf0, fbuf1)
    cbufs = (cbuf0, cbuf1)
    semfs = (semf0, semf1)
    semcs = (semc0, semc1)

    def issue(j):
        p = j & 1
        cf = pltpu.async_copy(feat_hbm.at[wid, j], fbufs[p], semfs[p])
        cc = pltpu.async_copy(centers_hbm.at[idx_v.at[j]], cbufs[p], semcs[p])
        return (cf, cc)

    out_row = wid * _NCH
    pending = {0: issue(0)}
    outcopies = []
    for j in range(_NCH):
        if j + 1 < _NCH:
            pending[j + 1] = issue(j + 1)
        for c in pending.pop(j):
            c.wait()
        p = j & 1
        fb = fbufs[p]
        cb = cbufs[p]

        @plsc.parallel_loop(0, _CH, unroll=_RU)
        def _row_loop(r, fb=fb, cb=cb, j=j):
            dot0 = jnp.zeros((_L,), jnp.float32)
            dot1 = jnp.zeros((_L,), jnp.float32)
            x20 = jnp.zeros((_L,), jnp.float32)
            x21 = jnp.zeros((_L,), jnp.float32)
            y20 = jnp.zeros((_L,), jnp.float32)
            y21 = jnp.zeros((_L,), jnp.float32)
            for q in range(_QR):
                f = fb[r, pl.ds(q * _L, _L)]
                c = cb[r, pl.ds(q * _L, _L)]
                if q & 1:
                    dot1 = dot1 + f * c
                    x21 = x21 + f * f
                    y21 = y21 + c * c
                else:
                    dot0 = dot0 + f * c
                    x20 = x20 + f * f
                    y20 = y20 + c * c
            off = (j * _CH + r) * _L
            dot_v[pl.ds(off, _L)] = dot0 + dot1
            x2_v[pl.ds(off, _L)] = x20 + x21
            y2_v[pl.ds(off, _L)] = y20 + y21

        sl = pl.ds(j * _PCH, _PCH)
        outcopies.append(
            pltpu.async_copy(dot_v.at[sl], dot_hbm.at[out_row + j], semo))
        outcopies.append(
            pltpu.async_copy(x2_v.at[sl], x2_hbm.at[out_row + j], semo))
        outcopies.append(
            pltpu.async_copy(y2_v.at[sl], y2_hbm.at[out_row + j], semo))
    for c in outcopies:
        c.wait()


def _sc_fused(feat4, centers, label3):
    mesh = plsc.VectorSubcoreMesh(core_axis_name="c", subcore_axis_name="s")
    out = jax.ShapeDtypeStruct((_NW * _NCH, _PCH), jnp.float32)
    fn = pl.kernel(
        _sc_fused_body,
        out_type=(out, out, out),
        mesh=mesh,
        scratch_types=[
            pltpu.VMEM((_NCH, _CH), jnp.int32),
            pltpu.VMEM((_CH, FEAT_DIM), jnp.float32),
            pltpu.VMEM((_CH, FEAT_DIM), jnp.float32),
            pltpu.VMEM((_CH, FEAT_DIM), jnp.float32),
            pltpu.VMEM((_CH, FEAT_DIM), jnp.float32),
            pltpu.VMEM((_BPW * _L,), jnp.float32),
            pltpu.VMEM((_BPW * _L,), jnp.float32),
            pltpu.VMEM((_BPW * _L,), jnp.float32),
            pltpu.SemaphoreType.DMA,
            pltpu.SemaphoreType.DMA,
            pltpu.SemaphoreType.DMA,
            pltpu.SemaphoreType.DMA,
            pltpu.SemaphoreType.DMA,
        ],
    )
    return fn(feat4, centers, label3)


_RK = BATCH * _L // 128


def _tc_finish_body(dp_ref, xp_ref, yp_ref, out_ref):
    ii = lax.broadcasted_iota(jnp.int32, (_RK, 128), 0)
    jj = lax.broadcasted_iota(jnp.int32, (_RK, 128), 1)
    bsel = (ii // _L == jj).astype(jnp.float32)
    dot = jnp.dot(dp_ref[...], bsel, preferred_element_type=jnp.float32)
    x2 = jnp.dot(xp_ref[...], bsel, preferred_element_type=jnp.float32)
    y2 = jnp.dot(yp_ref[...], bsel, preferred_element_type=jnp.float32)
    c = jnp.float32(CURVATURE)
    denom = 1.0 + c * x2 * y2 - 2.0 * c * dot + 1e-08
    a = 1.0 + c * y2
    b = 1.0 - c * x2
    num2 = a * a * x2 + b * b * y2 - 2.0 * a * b * dot
    num2 = jnp.maximum(num2, 0.0)
    sqrt_c = jnp.sqrt(c)
    norm = jnp.sqrt(num2) / denom
    z = jnp.clip(sqrt_c * norm, 1e-08, 1.0 - 1e-05)
    dist = (1.0 / sqrt_c) * jnp.log((1.0 + z) / (1.0 - z))
    out_ref[0, 0] = jnp.sum(dist) * jnp.float32(1.0 / BATCH)


def _tc_finish(dp2, xp2, yp2):
    return pl.pallas_call(
        _tc_finish_body,
        out_specs=pl.BlockSpec(memory_space=pltpu.SMEM),
        out_shape=jax.ShapeDtypeStruct((1, 1), jnp.float32),
    )(dp2, xp2, yp2)


def kernel(label, feat, centers):
    label3 = label.astype(jnp.int32).reshape(_NW, _NCH, _CH)
    feat4 = feat.reshape(_NW, _NCH, _CH, FEAT_DIM)
    dp, xp, yp = _sc_fused(feat4, centers, label3)
    loss = _tc_finish(dp, xp, yp)
    return loss[0, 0]
